# baseline (speedup 1.0000x reference)
import functools

import jax
import jax.numpy as jnp
from jax import lax
from jax.experimental import pallas as pl
from jax.experimental.pallas import tpu as pltpu
from jax.experimental.pallas import tpu_sc as plsc

_i32 = jnp.int32
_f32 = jnp.float32

H = 128
N_GRAPHS = 64
NC, NS, LANES = 2, 16, 16
NW = NC * NS
CH = 64
CE = 96
BN = 512


def _emb_gather(x_pad, emb):
    n_pad = x_pad.shape[0]
    rows_w = n_pad // NW
    nch = rows_w // CH
    mesh = plsc.VectorSubcoreMesh(core_axis_name="c", subcore_axis_name="s")

    @functools.partial(
        pl.kernel,
        out_type=jax.ShapeDtypeStruct((n_pad, H), _f32),
        mesh=mesh,
        compiler_params=pltpu.CompilerParams(needs_layout_passes=False),
        scratch_types=[
            pltpu.VMEM((CH,), _i32),
            pltpu.VMEM((CH, H), _f32),
            pltpu.SemaphoreType.DMA,
        ],
    )
    def k(x_hbm, emb_hbm, out_hbm, idx_v, rows_v, sem):
        wid = lax.axis_index("s") * NC + lax.axis_index("c")
        base = wid * rows_w
        for j in range(nch):
            pltpu.sync_copy(x_hbm.at[pl.ds(base + j * CH, CH)], idx_v)
            pltpu.async_copy(emb_hbm.at[idx_v], rows_v, sem).wait()
            pltpu.sync_copy(rows_v, out_hbm.at[pl.ds(base + j * CH, CH)])

    return k(x_pad, emb)


def _tc_attn(h, W, a_src, a_dst):
    n_pad = h.shape[0]

    def body(h_ref, w_ref, s_ref, d_ref, ht_ref, as_ref, ad_ref):
        ht = jnp.dot(h_ref[...], w_ref[...], preferred_element_type=_f32)
        ht_ref[...] = ht
        as_ref[...] = jnp.dot(ht, s_ref[...], preferred_element_type=_f32)
        ad_ref[...] = jnp.dot(ht, d_ref[...], preferred_element_type=_f32)

    return pl.pallas_call(
        body,
        grid=(n_pad // BN,),
        in_specs=[
            pl.BlockSpec((BN, H), lambda i: (i, 0)),
            pl.BlockSpec((H, H), lambda i: (0, 0)),
            pl.BlockSpec((H, 1), lambda i: (0, 0)),
            pl.BlockSpec((H, 1), lambda i: (0, 0)),
        ],
        out_specs=[
            pl.BlockSpec((BN, H), lambda i: (i, 0)),
            pl.BlockSpec((BN, 1), lambda i: (i, 0)),
            pl.BlockSpec((BN, 1), lambda i: (i, 0)),
        ],
        out_shape=[
            jax.ShapeDtypeStruct((n_pad, H), _f32),
            jax.ShapeDtypeStruct((n_pad, 1), _f32),
            jax.ShapeDtypeStruct((n_pad, 1), _f32),
        ],
    )(h, W, a_src.reshape(H, 1), a_dst.reshape(H, 1))


def _sc_edge(ht, a_s, a_d, src3, dst3):
    n_pad = ht.shape[0]
    nche = src3.shape[1] - 3
    rows_t = n_pad // NS
    mesh = plsc.VectorSubcoreMesh(core_axis_name="c", subcore_axis_name="s")

    @functools.partial(
        pl.kernel,
        out_type=[
            jax.ShapeDtypeStruct((NC, n_pad, H), _f32),
            jax.ShapeDtypeStruct((NC, n_pad), _f32),
        ],
        mesh=mesh,
        compiler_params=pltpu.CompilerParams(needs_layout_passes=False),
        scratch_types=[
            pltpu.VMEM((4, CE), _i32),
            pltpu.VMEM((4, CE), _i32),
            pltpu.VMEM((2, CE, H), _f32),
            pltpu.VMEM((n_pad,), _f32),
            pltpu.VMEM((n_pad,), _f32),
            pltpu.VMEM((2, CE), _f32),
            pltpu.VMEM_SHARED((n_pad, H), _f32),
            pltpu.VMEM_SHARED((n_pad,), _f32),
            pltpu.SemaphoreType.DMA,
            pltpu.SemaphoreType.DMA,
            pltpu.SemaphoreType.DMA,
        ],
    )
    def k(ht_hbm, as_hbm, ad_hbm, src_hbm, dst_hbm, u_hbm, den_hbm,
          srcs_v, dsts_v, rows_v, a_s_v, a_d_v, w_v, u_sh, den_sh,
          sem_i, sem_g, sem_s):
        cid = lax.axis_index("c")
        sid = lax.axis_index("s")
        wid = sid * NC + cid
        zero16 = jnp.zeros((LANES,), _f32)

        def issue_idx(g, q):
            pltpu.async_copy(src_hbm.at[wid, g], srcs_v.at[q], sem_i)
            pltpu.async_copy(dst_hbm.at[wid, g], dsts_v.at[q], sem_i)

        def wait_idx(q):
            pltpu.make_async_copy(src_hbm.at[wid, 0], srcs_v.at[q], sem_i).wait()
            pltpu.make_async_copy(dst_hbm.at[wid, 0], dsts_v.at[q], sem_i).wait()

        def zrow(i, _):
            for b in range(2):
                for c in range(H // LANES):
                    rows_v[b, i, pl.ds(c * LANES, LANES)] = zero16
            return 0
        lax.fori_loop(0, CE, zrow, 0)
        for b in range(2):
            for c in range(CE // LANES):
                w_v[b, pl.ds(c * LANES, LANES)] = zero16
        base = sid * rows_t
        for q in range(rows_t // CE):
            pltpu.sync_copy(rows_v.at[0], u_sh.at[pl.ds(base + q * CE, CE)])
            pltpu.sync_copy(w_v.at[0], den_sh.at[pl.ds(base + q * CE, CE)])
        rem = rows_t % CE
        if rem:
            off = base + (rows_t // CE) * CE
            pltpu.sync_copy(rows_v.at[0, pl.ds(0, rem)], u_sh.at[pl.ds(off, rem)])
            pltpu.sync_copy(w_v.at[0, pl.ds(0, rem)], den_sh.at[pl.ds(off, rem)])

        pltpu.sync_copy(as_hbm, a_s_v)
        pltpu.sync_copy(ad_hbm, a_d_v)
        issue_idx(0, 0)
        issue_idx(1, 1)
        issue_idx(2, 2)
        plsc.subcore_barrier()
        wait_idx(0)
        pltpu.async_copy(rows_v.at[1], u_sh.at[dsts_v.at[0]], sem_s, add=True)
        pltpu.async_copy(w_v.at[1], den_sh.at[dsts_v.at[0]], sem_s, add=True)
        pltpu.async_copy(ht_hbm.at[srcs_v.at[0]], rows_v.at[0], sem_g)

        def slot(g, _):
            p = jnp.bitwise_and(g, 1)
            q = jnp.bitwise_and(g, 3)
            qn = jnp.bitwise_and(g + 1, 3)
            pltpu.make_async_copy(ht_hbm.at[srcs_v.at[0]], rows_v.at[0], sem_g).wait()
            pltpu.make_async_copy(rows_v.at[0], u_sh.at[dsts_v.at[0]], sem_s).wait()
            pltpu.make_async_copy(w_v.at[0], den_sh.at[dsts_v.at[0]], sem_s).wait()
            issue_idx(g + 3, jnp.bitwise_and(g + 3, 3))
            wait_idx(0)
            pltpu.async_copy(ht_hbm.at[srcs_v.at[qn]], rows_v.at[1 - p], sem_g)
            for j in range(CE // LANES):
                s_idx = srcs_v[q, pl.ds(j * LANES, LANES)]
                d_idx = dsts_v[q, pl.ds(j * LANES, LANES)]
                e = (plsc.load_gather(a_s_v, [s_idx])
                     + plsc.load_gather(a_d_v, [d_idx]))
                w_v[p, pl.ds(j * LANES, LANES)] = jnp.exp(jnp.maximum(e, 0.2 * e))
            def scale(i, _):
                for t in range(2):
                    kk = 2 * i + t
                    k16 = jnp.full((LANES,), kk, _i32)
                    wk = plsc.load_gather(w_v.at[p], [k16])
                    for c in range(H // LANES):
                        rows_v[p, kk, pl.ds(c * LANES, LANES)] = (
                            rows_v[p, kk, pl.ds(c * LANES, LANES)] * wk)
                return 0
            lax.fori_loop(0, CE // 2, scale, 0)
            pltpu.async_copy(rows_v.at[p], u_sh.at[dsts_v.at[q]], sem_s, add=True)
            pltpu.async_copy(w_v.at[p], den_sh.at[dsts_v.at[q]], sem_s, add=True)
            return 0
        lax.fori_loop(0, nche, slot, 0)

        pltpu.make_async_copy(ht_hbm.at[srcs_v.at[0]], rows_v.at[0], sem_g).wait()
        pltpu.make_async_copy(rows_v.at[0], u_sh.at[dsts_v.at[0]], sem_s).wait()
        pltpu.make_async_copy(w_v.at[0], den_sh.at[dsts_v.at[0]], sem_s).wait()
        wait_idx(0)
        wait_idx(1)

        plsc.subcore_barrier()
        pltpu.sync_copy(u_sh.at[pl.ds(base, rows_t)],
                        u_hbm.at[cid, pl.ds(base, rows_t)])
        pltpu.sync_copy(den_sh.at[pl.ds(base, rows_t)],
                        den_hbm.at[cid, pl.ds(base, rows_t)])

    return k(ht, a_s, a_d, src3, dst3)


def _tc_post(u2, den2, gb, lw, lb):
    n_pad = u2.shape[1]

    def body(u_ref, den_ref, gb_ref, lw_ref, lb_ref, o_ref):
        u = u_ref[0] + u_ref[1]
        den = den_ref[0] + den_ref[1]
        agg = u / jnp.maximum(den, 1e-16) + gb_ref[...]
        hh = jnp.maximum(agg, 0.0)
        o_ref[...] = jnp.dot(hh, lw_ref[...], preferred_element_type=_f32) + lb_ref[...]

    return pl.pallas_call(
        body,
        grid=(n_pad // BN,),
        in_specs=[
            pl.BlockSpec((NC, BN, H), lambda i: (0, i, 0)),
            pl.BlockSpec((NC, BN, 1), lambda i: (0, i, 0)),
            pl.BlockSpec((1, H), lambda i: (0, 0)),
            pl.BlockSpec((H, H), lambda i: (0, 0)),
            pl.BlockSpec((1, H), lambda i: (0, 0)),
        ],
        out_specs=pl.BlockSpec((BN, H), lambda i: (i, 0)),
        out_shape=jax.ShapeDtypeStruct((n_pad, H), _f32),
    )(u2, den2.reshape(NC, n_pad, 1), gb.reshape(1, H), lw, lb.reshape(1, H))


def _tc_pool(h, batch2):
    n_pad = h.shape[0]

    def body(h_ref, b_ref, s_ref, c_ref):
        i = pl.program_id(0)
        onehot = (b_ref[...] == lax.broadcasted_iota(_i32, (BN, N_GRAPHS), 1)
                  ).astype(_f32)
        ps = lax.dot_general(onehot, h_ref[...], (((0,), (0,)), ((), ())),
                             preferred_element_type=_f32)
        pc = jnp.sum(onehot, axis=0).reshape(N_GRAPHS, 1)

        @pl.when(i == 0)
        def _():
            s_ref[...] = jnp.zeros(s_ref.shape, _f32)
            c_ref[...] = jnp.zeros(c_ref.shape, _f32)

        s_ref[...] += ps
        c_ref[...] += pc

    return pl.pallas_call(
        body,
        grid=(n_pad // BN,),
        in_specs=[
            pl.BlockSpec((BN, H), lambda i: (i, 0)),
            pl.BlockSpec((BN, 1), lambda i: (i, 0)),
        ],
        out_specs=[
            pl.BlockSpec((N_GRAPHS, H), lambda i: (0, 0)),
            pl.BlockSpec((N_GRAPHS, 1), lambda i: (0, 0)),
        ],
        out_shape=[
            jax.ShapeDtypeStruct((N_GRAPHS, H), _f32),
            jax.ShapeDtypeStruct((N_GRAPHS, 1), _f32),
        ],
    )(h, batch2)


def _tc_out(sums, counts, out_W, out_b):
    V = out_W.shape[1]
    BV = 640

    def body(s_ref, c_ref, w_ref, b_ref, o_ref):
        pooled = s_ref[...] / jnp.maximum(c_ref[...], 1.0)
        o_ref[...] = jnp.dot(pooled, w_ref[...], preferred_element_type=_f32) + b_ref[...]

    return pl.pallas_call(
        body,
        grid=(V // BV,),
        in_specs=[
            pl.BlockSpec((N_GRAPHS, H), lambda i: (0, 0)),
            pl.BlockSpec((N_GRAPHS, 1), lambda i: (0, 0)),
            pl.BlockSpec((H, BV), lambda i: (0, i)),
            pl.BlockSpec((1, BV), lambda i: (0, i)),
        ],
        out_specs=pl.BlockSpec((N_GRAPHS, BV), lambda i: (0, i)),
        out_shape=jax.ShapeDtypeStruct((N_GRAPHS, V), _f32),
    )(sums, counts, out_W, out_b.reshape(1, V))


def kernel(x, edge_index, batch, emb, gat_W, gat_a_src, gat_a_dst, gat_b,
           lin_W, lin_b, out_W, out_b):
    N = x.shape[0]
    E = edge_index.shape[1]
    layers = gat_W.shape[0]

    n_pad = -(-N // 2048) * 2048
    e_pad = -(-E // (4 * NW * CE)) * (4 * NW * CE)

    x_pad = jnp.concatenate([x.astype(_i32), jnp.zeros((n_pad - N,), _i32)])
    src = edge_index[0].astype(_i32)
    dst = edge_index[1].astype(_i32)
    pad_e = jnp.full((e_pad - E,), n_pad - 1, _i32)
    pad_chunk = jnp.full((NW, 3, CE), n_pad - 1, _i32)
    src3 = jnp.concatenate(
        [jnp.concatenate([src, pad_e]).reshape(NW, -1, CE), pad_chunk], axis=1)
    dst3 = jnp.concatenate(
        [jnp.concatenate([dst, pad_e]).reshape(NW, -1, CE), pad_chunk], axis=1)
    batch2 = jnp.concatenate(
        [batch.astype(_i32), jnp.full((n_pad - N,), N_GRAPHS, _i32)]
    ).reshape(n_pad, 1)

    h = _emb_gather(x_pad, emb)
    for l in range(layers):
        ht, a_s, a_d = _tc_attn(h, gat_W[l], gat_a_src[l], gat_a_dst[l])
        u2, den2 = _sc_edge(ht, a_s.reshape(-1), a_d.reshape(-1), src3, dst3)
        h = _tc_post(u2, den2, gat_b[l], lin_W[l], lin_b[l])
    sums, counts = _tc_pool(h, batch2)
    return _tc_out(sums, counts, out_W, out_b)


# speedup vs baseline: 1.3456x; 1.3456x over previous
import functools

import jax
import jax.numpy as jnp
from jax import lax
from jax.experimental import pallas as pl
from jax.experimental.pallas import tpu as pltpu
from jax.experimental.pallas import tpu_sc as plsc

_i32 = jnp.int32
_f32 = jnp.float32

H = 128
N_GRAPHS = 64
NC, NS, LANES = 2, 16, 16
NW = NC * NS
CH = 64
CE = 128
BN = 512


def _emb_gather(x_pad, emb):
    n_pad = x_pad.shape[0]
    rows_w = n_pad // NW
    nch = rows_w // CH
    mesh = plsc.VectorSubcoreMesh(core_axis_name="c", subcore_axis_name="s")

    @functools.partial(
        pl.kernel,
        out_type=jax.ShapeDtypeStruct((n_pad, H), _f32),
        mesh=mesh,
        compiler_params=pltpu.CompilerParams(needs_layout_passes=False),
        scratch_types=[
            pltpu.VMEM((CH,), _i32),
            pltpu.VMEM((CH, H), _f32),
            pltpu.SemaphoreType.DMA,
        ],
    )
    def k(x_hbm, emb_hbm, out_hbm, idx_v, rows_v, sem):
        wid = lax.axis_index("s") * NC + lax.axis_index("c")
        base = wid * rows_w
        for j in range(nch):
            pltpu.sync_copy(x_hbm.at[pl.ds(base + j * CH, CH)], idx_v)
            pltpu.async_copy(emb_hbm.at[idx_v], rows_v, sem).wait()
            pltpu.sync_copy(rows_v, out_hbm.at[pl.ds(base + j * CH, CH)])

    return k(x_pad, emb)


def _tc_attn(h, W, a_src, a_dst):
    n_pad = h.shape[0]

    def body(h_ref, w_ref, s_ref, d_ref, ht_ref, as_ref, ad_ref):
        ht = jnp.dot(h_ref[...], w_ref[...], preferred_element_type=_f32)
        ht_ref[...] = ht
        as_ref[...] = jnp.dot(ht, s_ref[...], preferred_element_type=_f32)
        ad_ref[...] = jnp.dot(ht, d_ref[...], preferred_element_type=_f32)

    return pl.pallas_call(
        body,
        grid=(n_pad // BN,),
        in_specs=[
            pl.BlockSpec((BN, H), lambda i: (i, 0)),
            pl.BlockSpec((H, H), lambda i: (0, 0)),
            pl.BlockSpec((H, 1), lambda i: (0, 0)),
            pl.BlockSpec((H, 1), lambda i: (0, 0)),
        ],
        out_specs=[
            pl.BlockSpec((BN, H), lambda i: (i, 0)),
            pl.BlockSpec((BN, 1), lambda i: (i, 0)),
            pl.BlockSpec((BN, 1), lambda i: (i, 0)),
        ],
        out_shape=[
            jax.ShapeDtypeStruct((n_pad, H), _f32),
            jax.ShapeDtypeStruct((n_pad, 1), _f32),
            jax.ShapeDtypeStruct((n_pad, 1), _f32),
        ],
    )(h, W, a_src.reshape(H, 1), a_dst.reshape(H, 1))


def _sc_edge(ht, a_s, a_d, src3, dst3):
    n_pad = ht.shape[0]
    nche = src3.shape[1]
    rows_t = n_pad // NS
    mesh = plsc.VectorSubcoreMesh(core_axis_name="c", subcore_axis_name="s")

    @functools.partial(
        pl.kernel,
        out_type=[
            jax.ShapeDtypeStruct((NC, n_pad, H), _f32),
            jax.ShapeDtypeStruct((NC, n_pad), _f32),
        ],
        mesh=mesh,
        compiler_params=pltpu.CompilerParams(needs_layout_passes=False),
        scratch_types=[
            pltpu.VMEM((n_pad,), _f32),
            pltpu.VMEM((n_pad,), _f32),
            pltpu.VMEM((1, CE), _i32),
            pltpu.VMEM((1, CE), _i32),
            pltpu.VMEM((CE, H), _f32),
            pltpu.VMEM((CE,), _f32),
            pltpu.VMEM_SHARED((n_pad, H), _f32),
            pltpu.VMEM_SHARED((n_pad,), _f32),
            pltpu.SemaphoreType.DMA,
        ],
    )
    def k(ht_hbm, as_hbm, ad_hbm, src_hbm, dst_hbm, u_hbm, den_hbm,
          a_s_v, a_d_v, src_v, dst_v, rows_v, w_v, u_sh, den_sh, sem):
        cid = lax.axis_index("c")
        sid = lax.axis_index("s")
        wid = sid * NC + cid
        zero16 = jnp.zeros((LANES,), _f32)

        def zrow(i, _):
            for c in range(H // LANES):
                rows_v[i, pl.ds(c * LANES, LANES)] = zero16
            return 0
        lax.fori_loop(0, CE, zrow, 0)
        for c in range(CE // LANES):
            w_v[pl.ds(c * LANES, LANES)] = zero16
        for q in range(rows_t // CE):
            pltpu.sync_copy(rows_v, u_sh.at[pl.ds(sid * rows_t + q * CE, CE)])
            pltpu.sync_copy(w_v, den_sh.at[pl.ds(sid * rows_t + q * CE, CE)])

        pltpu.sync_copy(as_hbm, a_s_v)
        pltpu.sync_copy(ad_hbm, a_d_v)
        plsc.subcore_barrier()

        def step(g, _):
            pltpu.sync_copy(src_hbm.at[wid, g], src_v.at[0])
            pltpu.sync_copy(dst_hbm.at[wid, g], dst_v.at[0])
            pltpu.async_copy(ht_hbm.at[src_v.at[0]], rows_v, sem).wait()
            for j in range(CE // LANES):
                s_idx = src_v[0, pl.ds(j * LANES, LANES)]
                d_idx = dst_v[0, pl.ds(j * LANES, LANES)]
                e = plsc.load_gather(a_s_v, [s_idx]) + plsc.load_gather(a_d_v, [d_idx])
                w_v[pl.ds(j * LANES, LANES)] = jnp.exp(jnp.maximum(e, 0.2 * e))

            def scale(kk, _):
                k16 = jnp.full((LANES,), kk, _i32)
                wk = plsc.load_gather(w_v, [k16])
                for c in range(H // LANES):
                    rows_v[kk, pl.ds(c * LANES, LANES)] = (
                        rows_v[kk, pl.ds(c * LANES, LANES)] * wk)
                return 0
            lax.fori_loop(0, CE, scale, 0)

            pltpu.sync_copy(rows_v, u_sh.at[dst_v.at[0]], add=True)
            pltpu.sync_copy(w_v, den_sh.at[dst_v.at[0]], add=True)
            return 0
        lax.fori_loop(0, nche, step, 0)

        plsc.subcore_barrier()
        pltpu.sync_copy(u_sh.at[pl.ds(sid * rows_t, rows_t)],
                        u_hbm.at[cid, pl.ds(sid * rows_t, rows_t)])
        pltpu.sync_copy(den_sh.at[pl.ds(sid * rows_t, rows_t)],
                        den_hbm.at[cid, pl.ds(sid * rows_t, rows_t)])

    return k(ht, a_s, a_d, src3, dst3)


def _tc_post(u2, den2, gb, lw, lb):
    n_pad = u2.shape[1]

    def body(u_ref, den_ref, gb_ref, lw_ref, lb_ref, o_ref):
        u = u_ref[0] + u_ref[1]
        den = den_ref[0] + den_ref[1]
        agg = u / jnp.maximum(den, 1e-16) + gb_ref[...]
        hh = jnp.maximum(agg, 0.0)
        o_ref[...] = jnp.dot(hh, lw_ref[...], preferred_element_type=_f32) + lb_ref[...]

    return pl.pallas_call(
        body,
        grid=(n_pad // BN,),
        in_specs=[
            pl.BlockSpec((NC, BN, H), lambda i: (0, i, 0)),
            pl.BlockSpec((NC, BN, 1), lambda i: (0, i, 0)),
            pl.BlockSpec((1, H), lambda i: (0, 0)),
            pl.BlockSpec((H, H), lambda i: (0, 0)),
            pl.BlockSpec((1, H), lambda i: (0, 0)),
        ],
        out_specs=pl.BlockSpec((BN, H), lambda i: (i, 0)),
        out_shape=jax.ShapeDtypeStruct((n_pad, H), _f32),
    )(u2, den2.reshape(NC, n_pad, 1), gb.reshape(1, H), lw, lb.reshape(1, H))


def _tc_pool(h, batch2):
    n_pad = h.shape[0]

    def body(h_ref, b_ref, s_ref, c_ref):
        i = pl.program_id(0)
        onehot = (b_ref[...] == lax.broadcasted_iota(_i32, (BN, N_GRAPHS), 1)
                  ).astype(_f32)
        ps = lax.dot_general(onehot, h_ref[...], (((0,), (0,)), ((), ())),
                             preferred_element_type=_f32)
        pc = jnp.sum(onehot, axis=0).reshape(N_GRAPHS, 1)

        @pl.when(i == 0)
        def _():
            s_ref[...] = jnp.zeros(s_ref.shape, _f32)
            c_ref[...] = jnp.zeros(c_ref.shape, _f32)

        s_ref[...] += ps
        c_ref[...] += pc

    return pl.pallas_call(
        body,
        grid=(n_pad // BN,),
        in_specs=[
            pl.BlockSpec((BN, H), lambda i: (i, 0)),
            pl.BlockSpec((BN, 1), lambda i: (i, 0)),
        ],
        out_specs=[
            pl.BlockSpec((N_GRAPHS, H), lambda i: (0, 0)),
            pl.BlockSpec((N_GRAPHS, 1), lambda i: (0, 0)),
        ],
        out_shape=[
            jax.ShapeDtypeStruct((N_GRAPHS, H), _f32),
            jax.ShapeDtypeStruct((N_GRAPHS, 1), _f32),
        ],
    )(h, batch2)


def _tc_out(sums, counts, out_W, out_b):
    V = out_W.shape[1]
    BV = 640

    def body(s_ref, c_ref, w_ref, b_ref, o_ref):
        pooled = s_ref[...] / jnp.maximum(c_ref[...], 1.0)
        o_ref[...] = jnp.dot(pooled, w_ref[...], preferred_element_type=_f32) + b_ref[...]

    return pl.pallas_call(
        body,
        grid=(V // BV,),
        in_specs=[
            pl.BlockSpec((N_GRAPHS, H), lambda i: (0, 0)),
            pl.BlockSpec((N_GRAPHS, 1), lambda i: (0, 0)),
            pl.BlockSpec((H, BV), lambda i: (0, i)),
            pl.BlockSpec((1, BV), lambda i: (0, i)),
        ],
        out_specs=pl.BlockSpec((N_GRAPHS, BV), lambda i: (0, i)),
        out_shape=jax.ShapeDtypeStruct((N_GRAPHS, V), _f32),
    )(sums, counts, out_W, out_b.reshape(1, V))


def kernel(x, edge_index, batch, emb, gat_W, gat_a_src, gat_a_dst, gat_b,
           lin_W, lin_b, out_W, out_b):
    N = x.shape[0]
    E = edge_index.shape[1]
    layers = gat_W.shape[0]

    n_pad = -(-N // 2048) * 2048
    e_pad = -(-E // (NW * CE)) * (NW * CE)

    x_pad = jnp.concatenate([x.astype(_i32), jnp.zeros((n_pad - N,), _i32)])
    src = edge_index[0].astype(_i32)
    dst = edge_index[1].astype(_i32)
    pad_e = jnp.full((e_pad - E,), n_pad - 1, _i32)
    src3 = jnp.concatenate([src, pad_e]).reshape(NW, -1, CE)
    dst3 = jnp.concatenate([dst, pad_e]).reshape(NW, -1, CE)
    batch2 = jnp.concatenate(
        [batch.astype(_i32), jnp.full((n_pad - N,), N_GRAPHS, _i32)]
    ).reshape(n_pad, 1)

    h = _emb_gather(x_pad, emb)
    for l in range(layers):
        ht, a_s, a_d = _tc_attn(h, gat_W[l], gat_a_src[l], gat_a_dst[l])
        u2, den2 = _sc_edge(ht, a_s.reshape(-1), a_d.reshape(-1), src3, dst3)
        h = _tc_post(u2, den2, gat_b[l], lin_W[l], lin_b[l])
    sums, counts = _tc_pool(h, batch2)
    return _tc_out(sums, counts, out_W, out_b)


# speedup vs baseline: 1.3860x; 1.0301x over previous
import functools

import jax
import jax.numpy as jnp
from jax import lax
from jax.experimental import pallas as pl
from jax.experimental.pallas import tpu as pltpu
from jax.experimental.pallas import tpu_sc as plsc

_i32 = jnp.int32
_f32 = jnp.float32

H = 128
N_GRAPHS = 64
NC, NS, LANES = 2, 16, 16
NW = NC * NS
CH = 64
CE = 128
BN = 512


def _emb_gather(x_pad, emb):
    n_pad = x_pad.shape[0]
    rows_w = n_pad // NW
    nch = rows_w // CH
    mesh = plsc.VectorSubcoreMesh(core_axis_name="c", subcore_axis_name="s")

    @functools.partial(
        pl.kernel,
        out_type=jax.ShapeDtypeStruct((n_pad, H), _f32),
        mesh=mesh,
        compiler_params=pltpu.CompilerParams(needs_layout_passes=False),
        scratch_types=[
            pltpu.VMEM((CH,), _i32),
            pltpu.VMEM((CH, H), _f32),
            pltpu.SemaphoreType.DMA,
        ],
    )
    def k(x_hbm, emb_hbm, out_hbm, idx_v, rows_v, sem):
        wid = lax.axis_index("s") * NC + lax.axis_index("c")
        base = wid * rows_w
        for j in range(nch):
            pltpu.sync_copy(x_hbm.at[pl.ds(base + j * CH, CH)], idx_v)
            pltpu.async_copy(emb_hbm.at[idx_v], rows_v, sem).wait()
            pltpu.sync_copy(rows_v, out_hbm.at[pl.ds(base + j * CH, CH)])

    return k(x_pad, emb)


def _tc_attn(h, W, a_src, a_dst):
    n_pad = h.shape[0]

    def body(h_ref, w_ref, s_ref, d_ref, ht_ref, as_ref, ad_ref):
        ht = jnp.dot(h_ref[...], w_ref[...], preferred_element_type=_f32)
        ht_ref[...] = ht
        as_ref[...] = jnp.dot(ht, s_ref[...], preferred_element_type=_f32)
        ad_ref[...] = jnp.dot(ht, d_ref[...], preferred_element_type=_f32)

    return pl.pallas_call(
        body,
        grid=(n_pad // BN,),
        in_specs=[
            pl.BlockSpec((BN, H), lambda i: (i, 0)),
            pl.BlockSpec((H, H), lambda i: (0, 0)),
            pl.BlockSpec((H, 1), lambda i: (0, 0)),
            pl.BlockSpec((H, 1), lambda i: (0, 0)),
        ],
        out_specs=[
            pl.BlockSpec((BN, H), lambda i: (i, 0)),
            pl.BlockSpec((BN, 1), lambda i: (i, 0)),
            pl.BlockSpec((BN, 1), lambda i: (i, 0)),
        ],
        out_shape=[
            jax.ShapeDtypeStruct((n_pad, H), _f32),
            jax.ShapeDtypeStruct((n_pad, 1), _f32),
            jax.ShapeDtypeStruct((n_pad, 1), _f32),
        ],
    )(h, W, a_src.reshape(H, 1), a_dst.reshape(H, 1))


def _sc_edge(ht, a_s, a_d, src3, dst3):
    n_pad = ht.shape[0]
    nche = src3.shape[1]
    rows_t = n_pad // NS
    mesh = plsc.VectorSubcoreMesh(core_axis_name="c", subcore_axis_name="s")

    @functools.partial(
        pl.kernel,
        out_type=[
            jax.ShapeDtypeStruct((NC, n_pad, H), _f32),
            jax.ShapeDtypeStruct((NC, n_pad), _f32),
        ],
        mesh=mesh,
        compiler_params=pltpu.CompilerParams(needs_layout_passes=False),
        scratch_types=[
            pltpu.VMEM((n_pad,), _f32),
            pltpu.VMEM((n_pad,), _f32),
            pltpu.VMEM((1, CE), _i32),
            pltpu.VMEM((1, CE), _i32),
            pltpu.VMEM((CE, H), _f32),
            pltpu.VMEM((CE,), _f32),
            pltpu.VMEM_SHARED((n_pad, H), _f32),
            pltpu.VMEM_SHARED((n_pad,), _f32),
            pltpu.SemaphoreType.DMA,
        ],
    )
    def k(ht_hbm, as_hbm, ad_hbm, src_hbm, dst_hbm, u_hbm, den_hbm,
          a_s_v, a_d_v, src_v, dst_v, rows_v, w_v, u_sh, den_sh, sem):
        cid = lax.axis_index("c")
        sid = lax.axis_index("s")
        wid = sid * NC + cid
        zero16 = jnp.zeros((LANES,), _f32)

        def zrow(i, _):
            for c in range(H // LANES):
                rows_v[i, pl.ds(c * LANES, LANES)] = zero16
            return 0
        lax.fori_loop(0, CE, zrow, 0)
        for c in range(CE // LANES):
            w_v[pl.ds(c * LANES, LANES)] = zero16
        for q in range(rows_t // CE):
            pltpu.sync_copy(rows_v, u_sh.at[pl.ds(sid * rows_t + q * CE, CE)])
            pltpu.sync_copy(w_v, den_sh.at[pl.ds(sid * rows_t + q * CE, CE)])

        pltpu.sync_copy(as_hbm, a_s_v)
        pltpu.sync_copy(ad_hbm, a_d_v)
        plsc.subcore_barrier()

        def step(g, _):
            pltpu.sync_copy(src_hbm.at[wid, g], src_v.at[0])
            pltpu.sync_copy(dst_hbm.at[wid, g], dst_v.at[0])
            gat = pltpu.async_copy(ht_hbm.at[src_v.at[0]], rows_v, sem)
            for j in range(CE // LANES):
                s_idx = src_v[0, pl.ds(j * LANES, LANES)]
                d_idx = dst_v[0, pl.ds(j * LANES, LANES)]
                e = plsc.load_gather(a_s_v, [s_idx]) + plsc.load_gather(a_d_v, [d_idx])
                w_v[pl.ds(j * LANES, LANES)] = jnp.exp(jnp.maximum(e, 0.2 * e))
            gat.wait()

            def scale(i, _):
                for t in range(2):
                    kk = 2 * i + t
                    k16 = jnp.full((LANES,), kk, _i32)
                    wk = plsc.load_gather(w_v, [k16])
                    for c in range(H // LANES):
                        rows_v[kk, pl.ds(c * LANES, LANES)] = (
                            rows_v[kk, pl.ds(c * LANES, LANES)] * wk)
                return 0
            lax.fori_loop(0, CE // 2, scale, 0)

            pltpu.sync_copy(rows_v, u_sh.at[dst_v.at[0]], add=True)
            pltpu.sync_copy(w_v, den_sh.at[dst_v.at[0]], add=True)
            return 0
        lax.fori_loop(0, nche, step, 0)

        plsc.subcore_barrier()
        pltpu.sync_copy(u_sh.at[pl.ds(sid * rows_t, rows_t)],
                        u_hbm.at[cid, pl.ds(sid * rows_t, rows_t)])
        pltpu.sync_copy(den_sh.at[pl.ds(sid * rows_t, rows_t)],
                        den_hbm.at[cid, pl.ds(sid * rows_t, rows_t)])

    return k(ht, a_s, a_d, src3, dst3)


def _tc_post(u2, den2, gb, lw, lb):
    n_pad = u2.shape[1]

    def body(u_ref, den_ref, gb_ref, lw_ref, lb_ref, o_ref):
        u = u_ref[0] + u_ref[1]
        den = den_ref[0] + den_ref[1]
        agg = u / jnp.maximum(den, 1e-16) + gb_ref[...]
        hh = jnp.maximum(agg, 0.0)
        o_ref[...] = jnp.dot(hh, lw_ref[...], preferred_element_type=_f32) + lb_ref[...]

    return pl.pallas_call(
        body,
        grid=(n_pad // BN,),
        in_specs=[
            pl.BlockSpec((NC, BN, H), lambda i: (0, i, 0)),
            pl.BlockSpec((NC, BN, 1), lambda i: (0, i, 0)),
            pl.BlockSpec((1, H), lambda i: (0, 0)),
            pl.BlockSpec((H, H), lambda i: (0, 0)),
            pl.BlockSpec((1, H), lambda i: (0, 0)),
        ],
        out_specs=pl.BlockSpec((BN, H), lambda i: (i, 0)),
        out_shape=jax.ShapeDtypeStruct((n_pad, H), _f32),
    )(u2, den2.reshape(NC, n_pad, 1), gb.reshape(1, H), lw, lb.reshape(1, H))


def _tc_pool(h, batch2):
    n_pad = h.shape[0]

    def body(h_ref, b_ref, s_ref, c_ref):
        i = pl.program_id(0)
        onehot = (b_ref[...] == lax.broadcasted_iota(_i32, (BN, N_GRAPHS), 1)
                  ).astype(_f32)
        ps = lax.dot_general(onehot, h_ref[...], (((0,), (0,)), ((), ())),
                             preferred_element_type=_f32)
        pc = jnp.sum(onehot, axis=0).reshape(N_GRAPHS, 1)

        @pl.when(i == 0)
        def _():
            s_ref[...] = jnp.zeros(s_ref.shape, _f32)
            c_ref[...] = jnp.zeros(c_ref.shape, _f32)

        s_ref[...] += ps
        c_ref[...] += pc

    return pl.pallas_call(
        body,
        grid=(n_pad // BN,),
        in_specs=[
            pl.BlockSpec((BN, H), lambda i: (i, 0)),
            pl.BlockSpec((BN, 1), lambda i: (i, 0)),
        ],
        out_specs=[
            pl.BlockSpec((N_GRAPHS, H), lambda i: (0, 0)),
            pl.BlockSpec((N_GRAPHS, 1), lambda i: (0, 0)),
        ],
        out_shape=[
            jax.ShapeDtypeStruct((N_GRAPHS, H), _f32),
            jax.ShapeDtypeStruct((N_GRAPHS, 1), _f32),
        ],
    )(h, batch2)


def _tc_out(sums, counts, out_W, out_b):
    V = out_W.shape[1]
    BV = 640

    def body(s_ref, c_ref, w_ref, b_ref, o_ref):
        pooled = s_ref[...] / jnp.maximum(c_ref[...], 1.0)
        o_ref[...] = jnp.dot(pooled, w_ref[...], preferred_element_type=_f32) + b_ref[...]

    return pl.pallas_call(
        body,
        grid=(V // BV,),
        in_specs=[
            pl.BlockSpec((N_GRAPHS, H), lambda i: (0, 0)),
            pl.BlockSpec((N_GRAPHS, 1), lambda i: (0, 0)),
            pl.BlockSpec((H, BV), lambda i: (0, i)),
            pl.BlockSpec((1, BV), lambda i: (0, i)),
        ],
        out_specs=pl.BlockSpec((N_GRAPHS, BV), lambda i: (0, i)),
        out_shape=jax.ShapeDtypeStruct((N_GRAPHS, V), _f32),
    )(sums, counts, out_W, out_b.reshape(1, V))


def kernel(x, edge_index, batch, emb, gat_W, gat_a_src, gat_a_dst, gat_b,
           lin_W, lin_b, out_W, out_b):
    N = x.shape[0]
    E = edge_index.shape[1]
    layers = gat_W.shape[0]

    n_pad = -(-N // 2048) * 2048
    e_pad = -(-E // (NW * CE)) * (NW * CE)

    x_pad = jnp.concatenate([x.astype(_i32), jnp.zeros((n_pad - N,), _i32)])
    src = edge_index[0].astype(_i32)
    dst = edge_index[1].astype(_i32)
    pad_e = jnp.full((e_pad - E,), n_pad - 1, _i32)
    src3 = jnp.concatenate([src, pad_e]).reshape(NW, -1, CE)
    dst3 = jnp.concatenate([dst, pad_e]).reshape(NW, -1, CE)
    batch2 = jnp.concatenate(
        [batch.astype(_i32), jnp.full((n_pad - N,), N_GRAPHS, _i32)]
    ).reshape(n_pad, 1)

    h = _emb_gather(x_pad, emb)
    for l in range(layers):
        ht, a_s, a_d = _tc_attn(h, gat_W[l], gat_a_src[l], gat_a_dst[l])
        u2, den2 = _sc_edge(ht, a_s.reshape(-1), a_d.reshape(-1), src3, dst3)
        h = _tc_post(u2, den2, gat_b[l], lin_W[l], lin_b[l])
    sums, counts = _tc_pool(h, batch2)
    return _tc_out(sums, counts, out_W, out_b)
